# baseline (speedup 1.0000x reference)
import functools

import jax
import jax.numpy as jnp
from jax import lax
from jax.experimental import pallas as pl
from jax.experimental.pallas import tpu as pltpu
from jax.experimental.pallas import tpu_sc as plsc

_BATCH = 16384
_STRIDE0 = 10000
_STRIDE1 = 100
_NC = 1
_NS = 16
_NW = _NC * _NS
_BPW = _BATCH // _NW


def _sc_lookup_body(xs_hbm, table_hbm, out_hbm, xs_v, idx_v, out_v, g_sem):
    wid = lax.axis_index("s") * _NC + lax.axis_index("c")
    base = wid * _BPW

    pltpu.sync_copy(xs_hbm.at[pl.ds(base * 3, _BPW * 3)], xs_v)

    for c in range(_BPW // 16):
        off = c * 16
        x0 = xs_v[pl.ds(off, 16)]
        x1 = xs_v[pl.ds(_BPW + off, 16)]
        x2 = xs_v[pl.ds(2 * _BPW + off, 16)]
        idx_v[pl.ds(off, 16)] = x0 * _STRIDE0 + x1 * _STRIDE1 + x2

    pltpu.async_copy(table_hbm.at[idx_v], out_v, g_sem).wait()
    pltpu.sync_copy(out_v, out_hbm.at[pl.ds(base, _BPW)])


@functools.partial(
    pl.kernel,
    out_type=jax.ShapeDtypeStruct((_BATCH,), jnp.float32),
    mesh=plsc.VectorSubcoreMesh(
        core_axis_name="c", subcore_axis_name="s", num_cores=_NC, num_subcores=_NS
    ),
    scratch_types=[
        pltpu.VMEM((_BPW * 3,), jnp.int32),
        pltpu.VMEM((_BPW,), jnp.int32),
        pltpu.VMEM((_BPW,), jnp.float32),
        pltpu.SemaphoreType.DMA,
    ],
)
def _sc_lookup(xs_hbm, table_hbm, out_hbm, xs_v, idx_v, out_v, g_sem):
    _sc_lookup_body(xs_hbm, table_hbm, out_hbm, xs_v, idx_v, out_v, g_sem)


def kernel(xs, embed_weight):
    xs_r = xs.reshape(_NW, _BPW, 3).transpose(0, 2, 1).reshape(-1)
    table = embed_weight.reshape(-1)
    return _sc_lookup(xs_r, table)


# speedup vs baseline: 1.0078x; 1.0078x over previous
import functools

import jax
import jax.numpy as jnp
from jax import lax
from jax.experimental import pallas as pl
from jax.experimental.pallas import tpu as pltpu
from jax.experimental.pallas import tpu_sc as plsc

_BATCH = 16384
_STRIDE0 = 10000
_STRIDE1 = 100
_NC = 1
_NS = 16
_NW = _NC * _NS
_BPW = _BATCH // _NW
_GCHUNK = 128
_NGATHER = _BPW // _GCHUNK
_CWORDS = 3 * _GCHUNK


def _sc_lookup_body(xs_hbm, table_hbm, out_hbm, xs_v, idx_v, out_v, in_sem, g_sem, o_sem):
    wid = lax.axis_index("s") * _NC + lax.axis_index("c")
    base = wid * _BPW

    in_copies = [
        pltpu.async_copy(
            xs_hbm.at[pl.ds((wid * _NGATHER + k) * _CWORDS, _CWORDS)],
            xs_v.at[k],
            in_sem.at[k],
        )
        for k in range(_NGATHER)
    ]

    gathers = []
    for k in range(_NGATHER):
        in_copies[k].wait()
        for c8 in range(_GCHUNK // 16):
            off = c8 * 16
            x0 = xs_v[k, pl.ds(off, 16)]
            x1 = xs_v[k, pl.ds(_GCHUNK + off, 16)]
            x2 = xs_v[k, pl.ds(2 * _GCHUNK + off, 16)]
            idx_v[k, pl.ds(off, 16)] = x0 * _STRIDE0 + x1 * _STRIDE1 + x2
        gathers.append(
            pltpu.async_copy(
                table_hbm.at[idx_v.at[k]],
                out_v.at[pl.ds(k * _GCHUNK, _GCHUNK)],
                g_sem.at[k],
            )
        )

    out_copies = []
    for k in range(_NGATHER):
        gathers[k].wait()
        out_copies.append(
            pltpu.async_copy(
                out_v.at[pl.ds(k * _GCHUNK, _GCHUNK)],
                out_hbm.at[pl.ds(base + k * _GCHUNK, _GCHUNK)],
                o_sem,
            )
        )
    for oc in out_copies:
        oc.wait()


@functools.partial(
    pl.kernel,
    out_type=jax.ShapeDtypeStruct((_BATCH,), jnp.float32),
    mesh=plsc.VectorSubcoreMesh(
        core_axis_name="c", subcore_axis_name="s", num_cores=_NC, num_subcores=_NS
    ),
    scratch_types=[
        pltpu.VMEM((_NGATHER, _CWORDS), jnp.int32),
        pltpu.VMEM((_NGATHER, _GCHUNK), jnp.int32),
        pltpu.VMEM((_BPW,), jnp.float32),
        pltpu.SemaphoreType.DMA((_NGATHER,)),
        pltpu.SemaphoreType.DMA((_NGATHER,)),
        pltpu.SemaphoreType.DMA,
    ],
)
def _sc_lookup(xs_hbm, table_hbm, out_hbm, xs_v, idx_v, out_v, in_sem, g_sem, o_sem):
    _sc_lookup_body(xs_hbm, table_hbm, out_hbm, xs_v, idx_v, out_v, in_sem, g_sem, o_sem)


def kernel(xs, embed_weight):
    xs_r = xs.reshape(_NW, _NGATHER, _GCHUNK, 3).transpose(0, 1, 3, 2).reshape(-1)
    table = embed_weight.reshape(-1)
    return _sc_lookup(xs_r, table)
